# baseline (speedup 1.0000x reference)
import functools

import jax
import jax.numpy as jnp
from jax import lax
from jax.experimental import pallas as pl
from jax.experimental.pallas import tpu as pltpu
from jax.experimental.pallas import tpu_sc as plsc

_NSP = 4
_A = 64
_L = 384
_N = 2048 * 64
_NW = 32
_BLK = 2048
_NS = 2
_NHALF = _N // _NS
_CH_H = _NHALF // _NW
_NCH_H = _CH_H // 128
_SPROWS_H = _NHALF // 128
_NPAD_H = _NHALF + _NSP * _BLK
_NB_H = _NPAD_H // _BLK

_MESH = dict(
    mesh=plsc.VectorSubcoreMesh(core_axis_name="c", subcore_axis_name="s"),
    compiler_params=pltpu.CompilerParams(needs_layout_passes=False),
)


def _wid():
    return lax.axis_index("s") * 2 + lax.axis_index("c")


@functools.partial(
    pl.kernel,
    out_type=jax.ShapeDtypeStruct((_NS * _NW, 16), jnp.int32),
    scratch_types=[
        pltpu.VMEM((_NCH_H, 128), jnp.int32),
        pltpu.VMEM((16,), jnp.int32),
    ],
    **_MESH,
)
def _ka(species_hbm, counts_hbm, sp_v, cnt_v):
    w = _wid()
    lane = lax.iota(jnp.int32, 16)

    def row(r, cnt):
        for l in range(8):
            v = sp_v[r, pl.ds(l * 16, 16)]
            for s in range(_NSP):
                c = plsc.all_reduce_population_count(v == s)
                cnt = jnp.where(lane == s, cnt + c, cnt)
        return cnt

    for h in range(_NS):
        pltpu.sync_copy(
            species_hbm.at[pl.ds(h * _SPROWS_H + w * _NCH_H, _NCH_H)], sp_v)
        cnt_v[...] = lax.fori_loop(0, _NCH_H, row, jnp.zeros((16,), jnp.int32))
        pltpu.sync_copy(cnt_v, counts_hbm.at[h * _NW + w])


def _make_kb(h):
    @functools.partial(
        pl.kernel,
        out_type=[
            jax.ShapeDtypeStruct((_NPAD_H, _L), jnp.float32),
            jax.ShapeDtypeStruct((_NHALF // 128, 128), jnp.int32),
        ],
        scratch_types=[
            pltpu.VMEM((_NCH_H, 128), jnp.int32),
            pltpu.VMEM((16,), jnp.int32),
            pltpu.VMEM((_NCH_H, 128), jnp.int32),
            pltpu.VMEM((2, 128, _L), jnp.float32),
            pltpu.SemaphoreType.DMA,
            pltpu.SemaphoreType.DMA,
        ],
        name=f"kb{h}",
        **_MESH,
    )
    def _kb(species_hbm, offs_hbm, aev_hbm, xs_hbm, dest_hbm,
            sp_v, off_v, dest_v, row_v, sem_in, sem_out):
        w = _wid()
        base_tok = h * _NHALF + w * _CH_H
        pltpu.sync_copy(
            species_hbm.at[pl.ds(h * _SPROWS_H + w * _NCH_H, _NCH_H)], sp_v)
        pltpu.sync_copy(offs_hbm.at[w], off_v)
        lane = lax.iota(jnp.int32, 16)

        pltpu.async_copy(aev_hbm.at[pl.ds(base_tok, 128)], row_v.at[0], sem_in)

        def chunk(r, cur):
            b = lax.rem(r, 2)
            for l in range(8):
                v = sp_v[r, pl.ds(l * 16, 16)]
                dest = jnp.zeros((16,), jnp.int32)
                for s in range(_NSP):
                    m = v == s
                    mi = m.astype(jnp.int32)
                    excl = plsc.cumsum(mi) - mi
                    c = plsc.all_reduce_population_count(m)
                    cur_s = jnp.sum(jnp.where(lane == s, cur, 0))
                    dest = jnp.where(m, cur_s + excl, dest)
                    cur = jnp.where(lane == s, cur + c, cur)
                dest_v[r, pl.ds(l * 16, 16)] = dest
            pltpu.make_async_copy(
                aev_hbm.at[pl.ds(base_tok + r * 128, 128)], row_v.at[b],
                sem_in).wait()
            pltpu.async_copy(row_v.at[b], xs_hbm.at[dest_v.at[r]], sem_out)

            @pl.when(r >= 1)
            def _():
                pltpu.make_async_copy(
                    row_v.at[1 - b], xs_hbm.at[dest_v.at[r - 1]],
                    sem_out).wait()

            @pl.when(r + 1 < _NCH_H)
            def _():
                pltpu.async_copy(
                    aev_hbm.at[pl.ds(base_tok + (r + 1) * 128, 128)],
                    row_v.at[1 - b], sem_in)

            return cur

        lax.fori_loop(0, _NCH_H, chunk, off_v[...])
        pltpu.make_async_copy(
            row_v.at[(_NCH_H - 1) % 2], xs_hbm.at[dest_v.at[_NCH_H - 1]],
            sem_out).wait()
        pltpu.sync_copy(dest_v, dest_hbm.at[pl.ds(w * _NCH_H, _NCH_H)])

    return _kb


_KB = tuple(_make_kb(h) for h in range(_NS))


def _celu(x):
    one = jnp.asarray(1.0, x.dtype)
    zero = jnp.asarray(0.0, x.dtype)
    alpha = jnp.asarray(0.1, x.dtype)
    ten = jnp.asarray(10.0, x.dtype)
    return jnp.where(x > zero, x, alpha * (jnp.exp(x * ten) - one))


def _mlp_body(bmap_ref, x_ref, w0_ref, b0_ref, w1_ref, b1_ref, w2_ref, b2_ref,
              w3_ref, b3_ref, y_ref):
    x = x_ref[...].astype(jnp.bfloat16)
    h = _celu((jnp.dot(x, w0_ref[0], preferred_element_type=jnp.float32)
               + b0_ref[0]).astype(jnp.bfloat16))
    h = _celu((jnp.dot(h, w1_ref[0], preferred_element_type=jnp.float32)
               + b1_ref[0]).astype(jnp.bfloat16))
    h = _celu((jnp.dot(h, w2_ref[0], preferred_element_type=jnp.float32)
               + b2_ref[0]).astype(jnp.bfloat16))
    y = jnp.dot(h, w3_ref[0], preferred_element_type=jnp.float32) + b3_ref[0]
    y_ref[...] = jnp.broadcast_to(y, (y.shape[0], 128))


def _ktc(bmap, xs, W0, b0, W1, b1, W2, b2, W3, b3):
    nb = xs.shape[0] // _BLK
    return pl.pallas_call(
        _mlp_body,
        grid_spec=pltpu.PrefetchScalarGridSpec(
            num_scalar_prefetch=1,
            grid=(nb,),
            in_specs=[
                pl.BlockSpec((_BLK, _L), lambda i, m: (i, 0)),
                pl.BlockSpec((1,) + W0.shape[1:], lambda i, m: (m[i], 0, 0)),
                pl.BlockSpec((1, 1) + b0.shape[2:], lambda i, m: (m[i], 0, 0)),
                pl.BlockSpec((1,) + W1.shape[1:], lambda i, m: (m[i], 0, 0)),
                pl.BlockSpec((1, 1) + b1.shape[2:], lambda i, m: (m[i], 0, 0)),
                pl.BlockSpec((1,) + W2.shape[1:], lambda i, m: (m[i], 0, 0)),
                pl.BlockSpec((1, 1) + b2.shape[2:], lambda i, m: (m[i], 0, 0)),
                pl.BlockSpec((1,) + W3.shape[1:], lambda i, m: (m[i], 0, 0)),
                pl.BlockSpec((1, 1) + b3.shape[2:], lambda i, m: (m[i], 0, 0)),
            ],
            out_specs=pl.BlockSpec((_BLK, 128), lambda i, m: (i, 0)),
        ),
        out_shape=jax.ShapeDtypeStruct((xs.shape[0], 128), jnp.float32),
        compiler_params=pltpu.CompilerParams(
            dimension_semantics=("arbitrary",),
        ),
    )(bmap, xs, W0, b0, W1, b1, W2, b2, W3, b3)


def _make_kc(h):
    nconf_w = _CH_H // _A

    @functools.partial(
        pl.kernel,
        out_type=jax.ShapeDtypeStruct((_NHALF // _A, 16), jnp.float32),
        scratch_types=[
            pltpu.VMEM((_NCH_H, 128), jnp.int32),
            pltpu.VMEM((2, 128, 128), jnp.float32),
            pltpu.VMEM((_CH_H // _A, 16), jnp.float32),
            pltpu.SemaphoreType.DMA,
        ],
        name=f"kc{h}",
        **_MESH,
    )
    def _kc(dest_hbm, y_hbm, out_hbm, dest_v, yrow_v, out_v, sem):
        w = _wid()
        pltpu.sync_copy(dest_hbm.at[pl.ds(w * _NCH_H, _NCH_H)], dest_v)
        pltpu.async_copy(y_hbm.at[dest_v.at[0]], yrow_v.at[0], sem)

        def chunk(r, carry):
            b = lax.rem(r, 2)
            pltpu.make_async_copy(
                y_hbm.at[dest_v.at[r]], yrow_v.at[b], sem).wait()

            @pl.when(r + 1 < _NCH_H)
            def _():
                pltpu.async_copy(
                    y_hbm.at[dest_v.at[r + 1]], yrow_v.at[1 - b], sem)

            for cc in range(2):
                acc = jnp.zeros((16,), jnp.float32)
                for k in range(_A):
                    acc = acc + yrow_v[b, cc * _A + k, pl.ds(0, 16)]
                out_v[r * 2 + cc, :] = acc
            return carry

        lax.fori_loop(0, _NCH_H, chunk, 0)
        pltpu.sync_copy(out_v, out_hbm.at[pl.ds(w * nconf_w, nconf_w)])

    return _kc


_KC = tuple(_make_kc(h) for h in range(_NS))


def _route_meta(cnt4):
    tot = cnt4.sum(axis=0)
    tot_r = ((tot + _BLK - 1) // _BLK) * _BLK
    bend = jnp.cumsum(tot_r)
    base = (bend - tot_r).astype(jnp.int32)
    excl_w = jnp.concatenate(
        [jnp.zeros((1, _NSP), jnp.int32), jnp.cumsum(cnt4, axis=0)[:-1]],
        axis=0)
    offs = jnp.pad(base[None, :] + excl_w, ((0, 0), (0, 16 - _NSP)))
    jb = jnp.arange(_NB_H, dtype=jnp.int32) * _BLK
    bmap = jnp.minimum((jb[:, None] >= bend[None, :]).sum(axis=1), _NSP - 1)
    return offs, bmap.astype(jnp.int32)


def kernel(species, aev, W0, b0, W1, b1, W2, b2, W3, b3):
    C, A, L = aev.shape
    species2d = species.reshape(_N // 128, 128).astype(jnp.int32)
    aev2d = aev.reshape(_N, L)
    wargs = (W0.astype(jnp.bfloat16), b0.reshape(_NSP, 1, -1),
             W1.astype(jnp.bfloat16), b1.reshape(_NSP, 1, -1),
             W2.astype(jnp.bfloat16), b2.reshape(_NSP, 1, -1),
             W3.astype(jnp.bfloat16), b3.reshape(_NSP, 1, -1))

    counts = _ka(species2d)
    meta = [_route_meta(counts[h * _NW:(h + 1) * _NW, :_NSP])
            for h in range(_NS)]
    routed = [_KB[h](species2d, meta[h][0], aev2d) for h in range(_NS)]
    ys = [_ktc(meta[h][1], routed[h][0], *wargs) for h in range(_NS)]
    outs = [_KC[h](routed[h][1], ys[h])[:, 0] for h in range(_NS)]
    return (species, jnp.concatenate(outs))


# speedup vs baseline: 1.0343x; 1.0343x over previous
import functools

import jax
import jax.numpy as jnp
from jax import lax
from jax.experimental import pallas as pl
from jax.experimental.pallas import tpu as pltpu
from jax.experimental.pallas import tpu_sc as plsc

_NSP = 4
_A = 64
_L = 384
_N = 2048 * 64
_NW = 32
_BLK = 4096
_NHALF = _N // 2
_CH_H = _NHALF // _NW
_NCH_H = _CH_H // 128
_SPROWS_H = _NHALF // 128
_NPAD_H = _NHALF + _NSP * _BLK
_NB_H = _NPAD_H // _BLK

_MESH = dict(
    mesh=plsc.VectorSubcoreMesh(core_axis_name="c", subcore_axis_name="s"),
    compiler_params=pltpu.CompilerParams(needs_layout_passes=False),
)


def _wid():
    return lax.axis_index("s") * 2 + lax.axis_index("c")


@functools.partial(
    pl.kernel,
    out_type=jax.ShapeDtypeStruct((2 * _NW, 16), jnp.int32),
    scratch_types=[
        pltpu.VMEM((_NCH_H, 128), jnp.int32),
        pltpu.VMEM((16,), jnp.int32),
    ],
    **_MESH,
)
def _ka(species_hbm, counts_hbm, sp_v, cnt_v):
    w = _wid()
    lane = lax.iota(jnp.int32, 16)

    def row(r, cnt):
        for l in range(8):
            v = sp_v[r, pl.ds(l * 16, 16)]
            for s in range(_NSP):
                c = plsc.all_reduce_population_count(v == s)
                cnt = jnp.where(lane == s, cnt + c, cnt)
        return cnt

    for h in range(2):
        pltpu.sync_copy(
            species_hbm.at[pl.ds(h * _SPROWS_H + w * _NCH_H, _NCH_H)], sp_v)
        cnt_v[...] = lax.fori_loop(0, _NCH_H, row, jnp.zeros((16,), jnp.int32))
        pltpu.sync_copy(cnt_v, counts_hbm.at[h * _NW + w])


def _make_kb(h):
    @functools.partial(
        pl.kernel,
        out_type=[
            jax.ShapeDtypeStruct((_NPAD_H, _L), jnp.float32),
            jax.ShapeDtypeStruct((_NHALF // 128, 128), jnp.int32),
        ],
        scratch_types=[
            pltpu.VMEM((_NCH_H, 128), jnp.int32),
            pltpu.VMEM((_NW, 16), jnp.int32),
            pltpu.VMEM((_NCH_H, 128), jnp.int32),
            pltpu.VMEM((2, 128, _L), jnp.float32),
            pltpu.SemaphoreType.DMA,
            pltpu.SemaphoreType.DMA,
        ],
        name=f"kb{h}",
        **_MESH,
    )
    def _kb(species_hbm, counts_hbm, aev_hbm, xs_hbm, dest_hbm,
            sp_v, cnts_v, dest_v, row_v, sem_in, sem_out):
        w = _wid()
        base_tok = h * _NHALF + w * _CH_H
        pltpu.sync_copy(
            species_hbm.at[pl.ds(h * _SPROWS_H + w * _NCH_H, _NCH_H)], sp_v)
        pltpu.sync_copy(counts_hbm.at[pl.ds(h * _NW, _NW)], cnts_v)
        lane = lax.iota(jnp.int32, 16)

        zero16 = jnp.zeros((16,), jnp.int32)

        def _acc(i, carry):
            t, e = carry
            row = cnts_v[i, :]
            iv = zero16 + i
            return (t + row, e + jnp.where(iv < w, row, 0))

        tot, excl = lax.fori_loop(0, _NW, _acc, (zero16, zero16))
        tot_r = ((tot + (_BLK - 1)) // _BLK) * _BLK
        bend = plsc.cumsum(tot_r)
        cur0 = (bend - tot_r) + excl

        pltpu.async_copy(aev_hbm.at[pl.ds(base_tok, 128)], row_v.at[0], sem_in)

        def chunk(r, cur):
            b = lax.rem(r, 2)
            for l in range(8):
                v = sp_v[r, pl.ds(l * 16, 16)]
                dest = jnp.zeros((16,), jnp.int32)
                for s in range(_NSP):
                    m = v == s
                    mi = m.astype(jnp.int32)
                    excl = plsc.cumsum(mi) - mi
                    c = plsc.all_reduce_population_count(m)
                    cur_s = jnp.sum(jnp.where(lane == s, cur, 0))
                    dest = jnp.where(m, cur_s + excl, dest)
                    cur = jnp.where(lane == s, cur + c, cur)
                dest_v[r, pl.ds(l * 16, 16)] = dest
            pltpu.make_async_copy(
                aev_hbm.at[pl.ds(base_tok + r * 128, 128)], row_v.at[b],
                sem_in).wait()
            pltpu.async_copy(row_v.at[b], xs_hbm.at[dest_v.at[r]], sem_out)

            @pl.when(r >= 1)
            def _():
                pltpu.make_async_copy(
                    row_v.at[1 - b], xs_hbm.at[dest_v.at[r - 1]],
                    sem_out).wait()

            @pl.when(r + 1 < _NCH_H)
            def _():
                pltpu.async_copy(
                    aev_hbm.at[pl.ds(base_tok + (r + 1) * 128, 128)],
                    row_v.at[1 - b], sem_in)

            return cur

        lax.fori_loop(0, _NCH_H, chunk, cur0)
        pltpu.make_async_copy(
            row_v.at[(_NCH_H - 1) % 2], xs_hbm.at[dest_v.at[_NCH_H - 1]],
            sem_out).wait()
        pltpu.sync_copy(dest_v, dest_hbm.at[pl.ds(w * _NCH_H, _NCH_H)])

    return _kb


_KB = (_make_kb(0), _make_kb(1))


def _celu(x):
    one = jnp.asarray(1.0, x.dtype)
    zero = jnp.asarray(0.0, x.dtype)
    alpha = jnp.asarray(0.1, x.dtype)
    ten = jnp.asarray(10.0, x.dtype)
    return jnp.where(x > zero, x, alpha * (jnp.exp(x * ten) - one))


def _mlp_body(bmap_ref, x_ref, w0_ref, b0_ref, w1_ref, b1_ref, w2_ref, b2_ref,
              w3_ref, b3_ref, y_ref):
    x = x_ref[...].astype(jnp.bfloat16)
    h = _celu((jnp.dot(x, w0_ref[0], preferred_element_type=jnp.float32)
               + b0_ref[0]).astype(jnp.bfloat16))
    h = _celu((jnp.dot(h, w1_ref[0], preferred_element_type=jnp.float32)
               + b1_ref[0]).astype(jnp.bfloat16))
    h = _celu((jnp.dot(h, w2_ref[0], preferred_element_type=jnp.float32)
               + b2_ref[0]).astype(jnp.bfloat16))
    y = jnp.dot(h, w3_ref[0], preferred_element_type=jnp.float32) + b3_ref[0]
    y_ref[...] = jnp.broadcast_to(y, (y.shape[0], 128))


def _ktc(bmap, xs, W0, b0, W1, b1, W2, b2, W3, b3):
    nb = xs.shape[0] // _BLK
    return pl.pallas_call(
        _mlp_body,
        grid_spec=pltpu.PrefetchScalarGridSpec(
            num_scalar_prefetch=1,
            grid=(nb,),
            in_specs=[
                pl.BlockSpec((_BLK, _L), lambda i, m: (i, 0)),
                pl.BlockSpec((1,) + W0.shape[1:], lambda i, m: (m[i], 0, 0)),
                pl.BlockSpec((1, 1) + b0.shape[2:], lambda i, m: (m[i], 0, 0)),
                pl.BlockSpec((1,) + W1.shape[1:], lambda i, m: (m[i], 0, 0)),
                pl.BlockSpec((1, 1) + b1.shape[2:], lambda i, m: (m[i], 0, 0)),
                pl.BlockSpec((1,) + W2.shape[1:], lambda i, m: (m[i], 0, 0)),
                pl.BlockSpec((1, 1) + b2.shape[2:], lambda i, m: (m[i], 0, 0)),
                pl.BlockSpec((1,) + W3.shape[1:], lambda i, m: (m[i], 0, 0)),
                pl.BlockSpec((1, 1) + b3.shape[2:], lambda i, m: (m[i], 0, 0)),
            ],
            out_specs=pl.BlockSpec((_BLK, 128), lambda i, m: (i, 0)),
        ),
        out_shape=jax.ShapeDtypeStruct((xs.shape[0], 128), jnp.float32),
        compiler_params=pltpu.CompilerParams(
            dimension_semantics=("arbitrary",),
        ),
    )(bmap, xs, W0, b0, W1, b1, W2, b2, W3, b3)


def _make_kc(h):
    nconf_w = _CH_H // _A

    @functools.partial(
        pl.kernel,
        out_type=jax.ShapeDtypeStruct((_NHALF // _A, 16), jnp.float32),
        scratch_types=[
            pltpu.VMEM((_NCH_H, 128), jnp.int32),
            pltpu.VMEM((2, 128, 128), jnp.float32),
            pltpu.VMEM((_CH_H // _A, 16), jnp.float32),
            pltpu.SemaphoreType.DMA,
        ],
        name=f"kc{h}",
        **_MESH,
    )
    def _kc(dest_hbm, y_hbm, out_hbm, dest_v, yrow_v, out_v, sem):
        w = _wid()
        pltpu.sync_copy(dest_hbm.at[pl.ds(w * _NCH_H, _NCH_H)], dest_v)
        pltpu.async_copy(y_hbm.at[dest_v.at[0]], yrow_v.at[0], sem)

        def chunk(r, carry):
            b = lax.rem(r, 2)
            pltpu.make_async_copy(
                y_hbm.at[dest_v.at[r]], yrow_v.at[b], sem).wait()

            @pl.when(r + 1 < _NCH_H)
            def _():
                pltpu.async_copy(
                    y_hbm.at[dest_v.at[r + 1]], yrow_v.at[1 - b], sem)

            for cc in range(2):
                acc = jnp.zeros((16,), jnp.float32)
                for k in range(_A):
                    acc = acc + yrow_v[b, cc * _A + k, pl.ds(0, 16)]
                out_v[r * 2 + cc, :] = acc
            return carry

        lax.fori_loop(0, _NCH_H, chunk, 0)
        pltpu.sync_copy(out_v, out_hbm.at[pl.ds(w * nconf_w, nconf_w)])

    return _kc


_KC = (_make_kc(0), _make_kc(1))


def _route_meta(cnt4):
    tot = cnt4.sum(axis=0)
    tot_r = ((tot + _BLK - 1) // _BLK) * _BLK
    bend = jnp.cumsum(tot_r)
    jb = jnp.arange(_NB_H, dtype=jnp.int32) * _BLK
    bmap = jnp.minimum((jb[:, None] >= bend[None, :]).sum(axis=1), _NSP - 1)
    return bmap.astype(jnp.int32)


def kernel(species, aev, W0, b0, W1, b1, W2, b2, W3, b3):
    C, A, L = aev.shape
    species2d = species.reshape(_N // 128, 128).astype(jnp.int32)
    aev2d = aev.reshape(_N, L)
    wargs = (W0.astype(jnp.bfloat16), b0.reshape(_NSP, 1, -1),
             W1.astype(jnp.bfloat16), b1.reshape(_NSP, 1, -1),
             W2.astype(jnp.bfloat16), b2.reshape(_NSP, 1, -1),
             W3.astype(jnp.bfloat16), b3.reshape(_NSP, 1, -1))

    counts = _ka(species2d)
    bmap0 = _route_meta(counts[:_NW, :_NSP])
    bmap1 = _route_meta(counts[_NW:, :_NSP])

    xs0, dest0 = _KB[0](species2d, counts, aev2d)
    xs1, dest1 = _KB[1](species2d, counts, aev2d)
    y0 = _ktc(bmap0, xs0, *wargs)
    y1 = _ktc(bmap1, xs1, *wargs)
    o0 = _KC[0](dest0, y0)
    o1 = _KC[1](dest1, y1)
    return (species, jnp.concatenate([o0[:, 0], o1[:, 0]]))
